# Initial kernel scaffold; baseline (speedup 1.0000x reference)
import jax
import jax.numpy as jnp
from jax import lax
from jax.experimental import pallas as pl
from jax.experimental.pallas import tpu as pltpu

NUM_EXPERTS = 64
TOPK = 8
ROUTER_SCALE = 2.5
BT = 512


def _gate_block(x_ref, wt_ref, b_ref, ow_ref, oi_ref):
    z = jnp.dot(x_ref[...], wt_ref[...],
                preferred_element_type=jnp.float32,
                precision=jax.lax.Precision.HIGHEST)
    z = z - jnp.max(z, axis=-1, keepdims=True)
    e = jnp.exp(z)
    scores = e / jnp.sum(e, axis=-1, keepdims=True)
    biased = scores + b_ref[...]

    iota = lax.broadcasted_iota(jnp.int32, (BT, NUM_EXPERTS), 1)
    vals = biased
    w_cols = []
    i_cols = []
    for _ in range(TOPK):
        m = jnp.max(vals, axis=-1, keepdims=True)
        idx = jnp.min(jnp.where(vals == m, iota, NUM_EXPERTS),
                      axis=-1, keepdims=True)
        sel = iota == idx
        w = jnp.sum(jnp.where(sel, scores, 0.0), axis=-1, keepdims=True)
        w_cols.append(w * ROUTER_SCALE)
        i_cols.append(idx)
        vals = jnp.where(sel, -jnp.inf, vals)
    ow_ref[...] = jnp.concatenate(w_cols, axis=1)
    oi_ref[...] = jnp.concatenate(i_cols, axis=1)


@jax.jit
def kernel(x, W, b):
    tokens, hidden = x.shape
    wt = W.T
    b2 = b.reshape(1, NUM_EXPERTS)
    grid = (tokens // BT,)
    ow, oi = pl.pallas_call(
        _gate_block,
        grid=grid,
        in_specs=[
            pl.BlockSpec((BT, hidden), lambda i: (i, 0)),
            pl.BlockSpec((hidden, NUM_EXPERTS), lambda i: (0, 0)),
            pl.BlockSpec((1, NUM_EXPERTS), lambda i: (0, 0)),
        ],
        out_specs=[
            pl.BlockSpec((BT, TOPK), lambda i: (i, 0)),
            pl.BlockSpec((BT, TOPK), lambda i: (i, 0)),
        ],
        out_shape=[
            jax.ShapeDtypeStruct((tokens, TOPK), jnp.float32),
            jax.ShapeDtypeStruct((tokens, TOPK), jnp.int32),
        ],
        compiler_params=pltpu.CompilerParams(
            dimension_semantics=("arbitrary",),
        ),
    )(x, wt, b2)
    return ow.astype(x.dtype), oi


# speedup vs baseline: 1.3909x; 1.3909x over previous
import jax
import jax.numpy as jnp
from jax import lax
from jax.experimental import pallas as pl
from jax.experimental.pallas import tpu as pltpu

NUM_EXPERTS = 64
TOPK = 8
ROUTER_SCALE = 2.5
BT = 512


def _gate_block(x_ref, wt_ref, b_ref, ow_ref, oi_ref):
    z = jnp.dot(x_ref[...].astype(jnp.bfloat16),
                wt_ref[...].astype(jnp.bfloat16),
                preferred_element_type=jnp.float32)
    z = z - jnp.max(z, axis=-1, keepdims=True)
    e = jnp.exp(z)
    scores = e / jnp.sum(e, axis=-1, keepdims=True)
    biased = scores + b_ref[...]

    iota = lax.broadcasted_iota(jnp.int32, (BT, NUM_EXPERTS), 1)
    vals = biased
    w_cols = []
    i_cols = []
    for _ in range(TOPK):
        m = jnp.max(vals, axis=-1, keepdims=True)
        idx = jnp.min(jnp.where(vals == m, iota, NUM_EXPERTS),
                      axis=-1, keepdims=True)
        sel = iota == idx
        w = jnp.sum(jnp.where(sel, scores, 0.0), axis=-1, keepdims=True)
        w_cols.append(w * ROUTER_SCALE)
        i_cols.append(idx)
        vals = jnp.where(sel, -jnp.inf, vals)
    ow_ref[...] = jnp.concatenate(w_cols, axis=1)
    oi_ref[...] = jnp.concatenate(i_cols, axis=1)


@jax.jit
def kernel(x, W, b):
    tokens, hidden = x.shape
    wt = W.T
    b2 = b.reshape(1, NUM_EXPERTS)
    grid = (tokens // BT,)
    ow, oi = pl.pallas_call(
        _gate_block,
        grid=grid,
        in_specs=[
            pl.BlockSpec((BT, hidden), lambda i: (i, 0)),
            pl.BlockSpec((hidden, NUM_EXPERTS), lambda i: (0, 0)),
            pl.BlockSpec((1, NUM_EXPERTS), lambda i: (0, 0)),
        ],
        out_specs=[
            pl.BlockSpec((BT, TOPK), lambda i: (i, 0)),
            pl.BlockSpec((BT, TOPK), lambda i: (i, 0)),
        ],
        out_shape=[
            jax.ShapeDtypeStruct((tokens, TOPK), jnp.float32),
            jax.ShapeDtypeStruct((tokens, TOPK), jnp.int32),
        ],
        compiler_params=pltpu.CompilerParams(
            dimension_semantics=("arbitrary",),
        ),
    )(x, wt, b2)
    return ow.astype(x.dtype), oi


# speedup vs baseline: 1.5357x; 1.1041x over previous
import jax
import jax.numpy as jnp
from jax import lax
from jax.experimental import pallas as pl
from jax.experimental.pallas import tpu as pltpu

NUM_EXPERTS = 64
TOPK = 8
ROUTER_SCALE = 2.5
BT = 512


def _gate_block(x_ref, wt_ref, b_ref, ow_ref, oi_ref):
    z = jnp.dot(x_ref[...], wt_ref[...],
                preferred_element_type=jnp.float32,
                precision=jax.lax.Precision.DEFAULT)
    z = z - jnp.max(z, axis=-1, keepdims=True)
    e = jnp.exp(z)
    s = jnp.sum(e, axis=-1, keepdims=True)
    biased = e + b_ref[...] * s

    bi = lax.bitcast_convert_type(biased, jnp.int32)
    sortable = bi ^ (lax.shift_right_arithmetic(bi, 31) & jnp.int32(0x7FFFFFFF))
    inv_iota = jnp.int32(NUM_EXPERTS - 1) - lax.broadcasted_iota(
        jnp.int32, (BT, NUM_EXPERTS), 1)
    key = (sortable & jnp.int32(-NUM_EXPERTS)) | inv_iota

    w_cols = []
    i_cols = []
    for _ in range(TOPK):
        m = jnp.max(key, axis=-1, keepdims=True)
        i_cols.append(jnp.int32(NUM_EXPERTS - 1) - (m & jnp.int32(NUM_EXPERTS - 1)))
        sel = key == m
        w_cols.append(jnp.sum(jnp.where(sel, e, 0.0), axis=-1, keepdims=True))
        key = jnp.where(sel, jnp.int32(-2**31), key)
    scale = ROUTER_SCALE / s
    ow_ref[...] = jnp.concatenate(w_cols, axis=1) * scale
    oi_ref[...] = jnp.concatenate(i_cols, axis=1)


@jax.jit
def kernel(x, W, b):
    tokens, hidden = x.shape
    wt = W.T
    b2 = b.reshape(1, NUM_EXPERTS)
    grid = (tokens // BT,)
    ow, oi = pl.pallas_call(
        _gate_block,
        grid=grid,
        in_specs=[
            pl.BlockSpec((BT, hidden), lambda i: (i, 0)),
            pl.BlockSpec((hidden, NUM_EXPERTS), lambda i: (0, 0)),
            pl.BlockSpec((1, NUM_EXPERTS), lambda i: (0, 0)),
        ],
        out_specs=[
            pl.BlockSpec((BT, TOPK), lambda i: (i, 0)),
            pl.BlockSpec((BT, TOPK), lambda i: (i, 0)),
        ],
        out_shape=[
            jax.ShapeDtypeStruct((tokens, TOPK), jnp.float32),
            jax.ShapeDtypeStruct((tokens, TOPK), jnp.int32),
        ],
        compiler_params=pltpu.CompilerParams(
            dimension_semantics=("arbitrary",),
        ),
    )(x, wt, b2)
    return ow.astype(x.dtype), oi


# speedup vs baseline: 1.6606x; 1.0813x over previous
import jax
import jax.numpy as jnp
from jax import lax
from jax.experimental import pallas as pl
from jax.experimental.pallas import tpu as pltpu

NUM_EXPERTS = 64
TOPK = 8
ROUTER_SCALE = 2.5
BT = 512
CHUNK = 64


def _gate_block(x_ref, wt_ref, b_ref, ow_ref, oi_ref):
    z = jnp.dot(x_ref[...], wt_ref[...],
                preferred_element_type=jnp.float32,
                precision=jax.lax.Precision.DEFAULT)
    z = z - jnp.max(z, axis=-1, keepdims=True)
    e = jnp.exp(z)
    s = jnp.sum(e, axis=-1, keepdims=True)
    biased = e + b_ref[...] * s

    bi = lax.bitcast_convert_type(biased, jnp.int32)
    sortable = bi ^ (lax.shift_right_arithmetic(bi, 31) & jnp.int32(0x7FFFFFFF))
    inv_iota = jnp.int32(NUM_EXPERTS - 1) - lax.broadcasted_iota(
        jnp.int32, (BT, NUM_EXPERTS), 1)
    packed = ((lax.shift_right_arithmetic(sortable, 1) & jnp.int32(-NUM_EXPERTS))
              | inv_iota) + jnp.int32(0x40000000)
    keyf = lax.bitcast_convert_type(packed, jnp.float32)

    scale = ROUTER_SCALE / s

    for r in range(BT // CHUNK):
        rows = slice(r * CHUNK, (r + 1) * CHUNK)
        ck = keyf[rows, :]
        ce = e[rows, :]
        w_cols = []
        i_cols = []
        for _ in range(TOPK):
            m = jnp.max(ck, axis=-1, keepdims=True)
            mi = lax.bitcast_convert_type(m, jnp.int32)
            i_cols.append(jnp.int32(NUM_EXPERTS - 1)
                          - (mi & jnp.int32(NUM_EXPERTS - 1)))
            sel = ck == m
            w_cols.append(jnp.sum(jnp.where(sel, ce, 0.0),
                                  axis=-1, keepdims=True))
            ck = jnp.where(sel, 0.0, ck)
        ow_ref[rows, :] = jnp.concatenate(w_cols, axis=1) * scale[rows, :]
        oi_ref[rows, :] = jnp.concatenate(i_cols, axis=1)


@jax.jit
def kernel(x, W, b):
    tokens, hidden = x.shape
    wt = W.T
    b2 = b.reshape(1, NUM_EXPERTS)
    grid = (tokens // BT,)
    ow, oi = pl.pallas_call(
        _gate_block,
        grid=grid,
        in_specs=[
            pl.BlockSpec((BT, hidden), lambda i: (i, 0)),
            pl.BlockSpec((hidden, NUM_EXPERTS), lambda i: (0, 0)),
            pl.BlockSpec((1, NUM_EXPERTS), lambda i: (0, 0)),
        ],
        out_specs=[
            pl.BlockSpec((BT, TOPK), lambda i: (i, 0)),
            pl.BlockSpec((BT, TOPK), lambda i: (i, 0)),
        ],
        out_shape=[
            jax.ShapeDtypeStruct((tokens, TOPK), jnp.float32),
            jax.ShapeDtypeStruct((tokens, TOPK), jnp.int32),
        ],
        compiler_params=pltpu.CompilerParams(
            dimension_semantics=("arbitrary",),
        ),
    )(x, wt, b2)
    return ow.astype(x.dtype), oi


# speedup vs baseline: 1.8406x; 1.1084x over previous
import jax
import jax.numpy as jnp
from jax import lax
from jax.experimental import pallas as pl
from jax.experimental.pallas import tpu as pltpu

NUM_EXPERTS = 64
TOPK = 8
ROUTER_SCALE = 2.5
BT = 1024
CHUNK = 64


def _gate_block(x_ref, wt_ref, b_ref, ow_ref, oi_ref):
    z = jnp.dot(x_ref[...], wt_ref[...],
                preferred_element_type=jnp.float32,
                precision=jax.lax.Precision.DEFAULT)
    z = z - jnp.max(z, axis=-1, keepdims=True)
    e = jnp.exp(z)
    s = jnp.sum(e, axis=-1, keepdims=True)
    biased = e + b_ref[...] * s

    bi = lax.bitcast_convert_type(biased, jnp.int32)
    sortable = bi ^ (lax.shift_right_arithmetic(bi, 31) & jnp.int32(0x7FFFFFFF))
    inv_iota = jnp.int32(NUM_EXPERTS - 1) - lax.broadcasted_iota(
        jnp.int32, (BT, NUM_EXPERTS), 1)
    packed = ((lax.shift_right_arithmetic(sortable, 1) & jnp.int32(-NUM_EXPERTS))
              | inv_iota) + jnp.int32(0x40000000)
    keyf = lax.bitcast_convert_type(packed, jnp.float32)

    scale = ROUTER_SCALE / s

    for r in range(BT // CHUNK):
        rows = slice(r * CHUNK, (r + 1) * CHUNK)
        ck = keyf[rows, :]
        ce = e[rows, :]
        w_cols = []
        i_cols = []
        for _ in range(TOPK):
            m = jnp.max(ck, axis=-1, keepdims=True)
            mi = lax.bitcast_convert_type(m, jnp.int32)
            i_cols.append(jnp.int32(NUM_EXPERTS - 1)
                          - (mi & jnp.int32(NUM_EXPERTS - 1)))
            sel = ck == m
            w_cols.append(jnp.sum(jnp.where(sel, ce, 0.0),
                                  axis=-1, keepdims=True))
            ck = jnp.where(sel, 0.0, ck)
        ow_ref[rows, :] = jnp.concatenate(w_cols, axis=1) * scale[rows, :]
        oi_ref[rows, :] = jnp.concatenate(i_cols, axis=1)


@jax.jit
def kernel(x, W, b):
    tokens, hidden = x.shape
    wt = W.T
    b2 = b.reshape(1, NUM_EXPERTS)
    grid = (tokens // BT,)
    ow, oi = pl.pallas_call(
        _gate_block,
        grid=grid,
        in_specs=[
            pl.BlockSpec((BT, hidden), lambda i: (i, 0)),
            pl.BlockSpec((hidden, NUM_EXPERTS), lambda i: (0, 0)),
            pl.BlockSpec((1, NUM_EXPERTS), lambda i: (0, 0)),
        ],
        out_specs=[
            pl.BlockSpec((BT, TOPK), lambda i: (i, 0)),
            pl.BlockSpec((BT, TOPK), lambda i: (i, 0)),
        ],
        out_shape=[
            jax.ShapeDtypeStruct((tokens, TOPK), jnp.float32),
            jax.ShapeDtypeStruct((tokens, TOPK), jnp.int32),
        ],
        compiler_params=pltpu.CompilerParams(
            dimension_semantics=("arbitrary",),
        ),
    )(x, wt, b2)
    return ow.astype(x.dtype), oi
